# baseline (speedup 1.0000x reference)
import functools

import jax
import jax.numpy as jnp
from jax import lax
from jax.experimental import pallas as pl
from jax.experimental.pallas import tpu as pltpu
from jax.experimental.pallas import tpu_sc as plsc

_B = 16384
_NUMERIC_DIM = 16
_EMBED_DIM = 32
_OUTPUT_DIM = 128

_NC = 2
_NS = 16
_NW = _NC * _NS
_BPW = _B // _NW

@functools.cache
def _make_sc_gather3():
    mesh = plsc.VectorSubcoreMesh(core_axis_name="c", subcore_axis_name="s")

    @functools.partial(
        pl.kernel,
        out_type=(
            jax.ShapeDtypeStruct((_B, _EMBED_DIM), jnp.float32),
            jax.ShapeDtypeStruct((_B, _EMBED_DIM), jnp.float32),
            jax.ShapeDtypeStruct((_B, _EMBED_DIM), jnp.float32),
        ),
        mesh=mesh,
        scratch_types=[
            pltpu.VMEM((_BPW,), jnp.int32),
            pltpu.VMEM((_BPW,), jnp.int32),
            pltpu.VMEM((_BPW,), jnp.int32),
            pltpu.VMEM((_BPW, _EMBED_DIM), jnp.float32),
            pltpu.VMEM((_BPW, _EMBED_DIM), jnp.float32),
            pltpu.VMEM((_BPW, _EMBED_DIM), jnp.float32),
            pltpu.SemaphoreType.DMA,
            pltpu.SemaphoreType.DMA,
            pltpu.SemaphoreType.DMA,
        ],
        compiler_params=pltpu.CompilerParams(use_tc_tiling_on_sc=False),
    )
    def _sc_gather3(cat_idx_hbm, host_idx_hbm, dom_idx_hbm,
                    cat_tab_hbm, host_tab_hbm, dom_tab_hbm,
                    cat_out, host_out, dom_out,
                    ci_v, hi_v, di_v, cr_v, hr_v, dr_v, s0, s1, s2):
        wid = lax.axis_index("s") * _NC + lax.axis_index("c")
        base = wid * _BPW
        pltpu.sync_copy(cat_idx_hbm.at[pl.ds(base, _BPW)], ci_v)
        pltpu.sync_copy(host_idx_hbm.at[pl.ds(base, _BPW)], hi_v)
        pltpu.sync_copy(dom_idx_hbm.at[pl.ds(base, _BPW)], di_v)
        c0 = pltpu.async_copy(cat_tab_hbm.at[ci_v], cr_v, s0)
        c1 = pltpu.async_copy(host_tab_hbm.at[hi_v], hr_v, s1)
        c2 = pltpu.async_copy(dom_tab_hbm.at[di_v], dr_v, s2)
        c0.wait()
        c1.wait()
        c2.wait()
        pltpu.sync_copy(cr_v, cat_out.at[pl.ds(base, _BPW)])
        pltpu.sync_copy(hr_v, host_out.at[pl.ds(base, _BPW)])
        pltpu.sync_copy(dr_v, dom_out.at[pl.ds(base, _BPW)])

    return _sc_gather3


_ROWS = 2048


def _tc_body(num_ref, cat_ref, host_ref, dom_ref, w_ref, b_ref, g_ref,
             be_ref, out_ref):
    x = jnp.concatenate(
        [num_ref[...], cat_ref[...], host_ref[...], dom_ref[...]], axis=-1)
    h = jnp.dot(x, w_ref[...], preferred_element_type=jnp.float32) + b_ref[...]
    mean = jnp.mean(h, axis=-1, keepdims=True)
    var = jnp.mean(jnp.square(h - mean), axis=-1, keepdims=True)
    y = (h - mean) * lax.rsqrt(var + 1e-5) * g_ref[...] + be_ref[...]
    out_ref[...] = y * 0.5 * (1.0 + lax.erf(y * 0.7071067811865476))


def _tc_dense(meta_numeric, cat_emb, host_emb, dom_emb, W, b, gamma, beta):
    grid = _B // _ROWS
    return pl.pallas_call(
        _tc_body,
        grid=(grid,),
        in_specs=[
            pl.BlockSpec((_ROWS, _NUMERIC_DIM), lambda i: (i, 0)),
            pl.BlockSpec((_ROWS, _EMBED_DIM), lambda i: (i, 0)),
            pl.BlockSpec((_ROWS, _EMBED_DIM), lambda i: (i, 0)),
            pl.BlockSpec((_ROWS, _EMBED_DIM), lambda i: (i, 0)),
            pl.BlockSpec((_NUMERIC_DIM + 3 * _EMBED_DIM, _OUTPUT_DIM),
                         lambda i: (0, 0)),
            pl.BlockSpec((1, _OUTPUT_DIM), lambda i: (0, 0)),
            pl.BlockSpec((1, _OUTPUT_DIM), lambda i: (0, 0)),
            pl.BlockSpec((1, _OUTPUT_DIM), lambda i: (0, 0)),
        ],
        out_specs=pl.BlockSpec((_ROWS, _OUTPUT_DIM), lambda i: (i, 0)),
        out_shape=jax.ShapeDtypeStruct((_B, _OUTPUT_DIM), jnp.float32),
    )(meta_numeric, cat_emb, host_emb, dom_emb, W,
      b.reshape(1, _OUTPUT_DIM), gamma.reshape(1, _OUTPUT_DIM),
      beta.reshape(1, _OUTPUT_DIM))


def kernel(meta_numeric, meta_category_id, meta_host_id, meta_domain_id,
           cat_table, host_table, domain_table, W, b, gamma, beta):
    cat_emb, host_emb, dom_emb = _make_sc_gather3()(
        meta_category_id.astype(jnp.int32),
        meta_host_id.astype(jnp.int32),
        meta_domain_id.astype(jnp.int32),
        cat_table, host_table, domain_table)
    return _tc_dense(meta_numeric, cat_emb, host_emb, dom_emb,
                     W, b, gamma, beta)
